# baseline (speedup 1.0000x reference)
import jax
import jax.numpy as jnp
from jax.experimental import pallas as pl
from jax.experimental.pallas import tpu as pltpu

_BD = 512


def _fused_kernel(samples_ref, enc_ref, cent_ref, out_ref, t_ref):
    i = pl.program_id(0)
    n = pl.num_programs(0)

    part = jax.lax.dot_general(
        cent_ref[...], enc_ref[...],
        (((1,), (0,)), ((), ())),
        preferred_element_type=jnp.float32,
    )

    @pl.when(i == 0)
    def _init():
        t_ref[...] = part

    @pl.when(i > 0)
    def _acc():
        t_ref[...] += part

    @pl.when(i == n - 1)
    def _final():
        out_ref[...] = jax.lax.dot_general(
            samples_ref[...], t_ref[...],
            (((1,), (1,)), ((), ())),
            preferred_element_type=jnp.float32,
        )


def kernel(samples, enc_weight, cent_weight):
    batch, n_features = samples.shape
    n_classes, n_dims = cent_weight.shape
    grid = (n_dims // _BD,)
    return pl.pallas_call(
        _fused_kernel,
        grid=grid,
        in_specs=[
            pl.BlockSpec((batch, n_features), lambda i: (0, 0)),
            pl.BlockSpec((_BD, n_features), lambda i: (i, 0)),
            pl.BlockSpec((n_classes, _BD), lambda i: (0, i)),
        ],
        out_specs=pl.BlockSpec((batch, n_classes), lambda i: (0, 0)),
        scratch_shapes=[pltpu.VMEM((n_classes, n_features), jnp.float32)],
        out_shape=jax.ShapeDtypeStruct((batch, n_classes), jnp.float32),
    )(samples, enc_weight, cent_weight)


# speedup vs baseline: 1.2967x; 1.2967x over previous
import jax
import jax.numpy as jnp
from jax.experimental import pallas as pl


def _fused_kernel(samples_ref, enc_ref, cent_ref, out_ref):
    t = jax.lax.dot_general(
        cent_ref[...], enc_ref[...],
        (((1,), (0,)), ((), ())),
        preferred_element_type=jnp.float32,
    )
    out_ref[...] = jax.lax.dot_general(
        samples_ref[...], t,
        (((1,), (1,)), ((), ())),
        preferred_element_type=jnp.float32,
    )


def kernel(samples, enc_weight, cent_weight):
    batch, n_features = samples.shape
    n_classes = cent_weight.shape[0]
    return pl.pallas_call(
        _fused_kernel,
        out_shape=jax.ShapeDtypeStruct((batch, n_classes), jnp.float32),
    )(samples, enc_weight, cent_weight)


# speedup vs baseline: 1.3120x; 1.0118x over previous
import jax
import jax.numpy as jnp
from jax.experimental import pallas as pl
from jax.experimental.pallas import tpu as pltpu

_NCHUNK = 8


def _fused_kernel(s_hbm, e_hbm, c_hbm, out_ref,
                  s_v, e_v, c_v, sem_e, sem_s, sem_c):
    d_total = e_hbm.shape[0]
    ch = d_total // _NCHUNK

    copies_e = []
    for i in range(_NCHUNK):
        cp = pltpu.make_async_copy(
            e_hbm.at[pl.ds(i * ch, ch), :],
            e_v.at[pl.ds(i * ch, ch), :],
            sem_e.at[i],
        )
        cp.start()
        copies_e.append(cp)
    cp_c = pltpu.make_async_copy(c_hbm, c_v, sem_c)
    cp_c.start()
    cp_s = pltpu.make_async_copy(s_hbm, s_v, sem_s)
    cp_s.start()

    cp_c.wait()
    t = None
    for i in range(_NCHUNK):
        copies_e[i].wait()
        part = jax.lax.dot_general(
            c_v[:, i * ch:(i + 1) * ch],
            e_v[i * ch:(i + 1) * ch, :],
            (((1,), (0,)), ((), ())),
            preferred_element_type=jnp.float32,
        )
        t = part if t is None else t + part

    cp_s.wait()
    out_ref[...] = jax.lax.dot_general(
        s_v[...], t,
        (((1,), (1,)), ((), ())),
        preferred_element_type=jnp.float32,
    )


def kernel(samples, enc_weight, cent_weight):
    batch, n_features = samples.shape
    n_classes, n_dims = cent_weight.shape
    return pl.pallas_call(
        _fused_kernel,
        in_specs=[
            pl.BlockSpec(memory_space=pl.ANY),
            pl.BlockSpec(memory_space=pl.ANY),
            pl.BlockSpec(memory_space=pl.ANY),
        ],
        out_specs=pl.BlockSpec(memory_space=pltpu.VMEM),
        out_shape=jax.ShapeDtypeStruct((batch, n_classes), jnp.float32),
        scratch_shapes=[
            pltpu.VMEM((batch, n_features), jnp.float32),
            pltpu.VMEM((n_dims, n_features), jnp.float32),
            pltpu.VMEM((n_classes, n_dims), jnp.float32),
            pltpu.SemaphoreType.DMA((_NCHUNK,)),
            pltpu.SemaphoreType.DMA,
            pltpu.SemaphoreType.DMA,
        ],
    )(samples, enc_weight, cent_weight)
